# baseline (speedup 1.0000x reference)
import functools

import jax
import jax.numpy as jnp
from jax import lax
from jax.experimental import pallas as pl
from jax.experimental.pallas import tpu as pltpu
from jax.experimental.pallas import tpu_sc as plsc

B = 4096
H = 200
HP = 256
HQ = 64
HR = H - 3 * HQ
NV = 13
HV = NV * 16
D = 64
L = 16
ND = D // L
NBUF = 6


def kernel(indices, table):
    info = plsc.get_sparse_core_info()
    nw = info.num_cores * info.num_subcores
    bpw = B // nw
    idxp = jnp.pad(indices.astype(jnp.int32), ((0, 0), (0, HP - H)))
    pos = jnp.arange(HP, dtype=jnp.int32)[None, :]
    bb = jnp.arange(B, dtype=jnp.int32)[:, None]
    idxp = jnp.where(pos < H, idxp, (bb * 8 + pos) % (table.shape[0] - 1))
    idx4 = idxp.reshape(4 * B, HQ)

    mesh = plsc.VectorSubcoreMesh(core_axis_name="c", subcore_axis_name="s")

    @functools.partial(
        pl.kernel,
        out_type=jax.ShapeDtypeStruct((B, D), jnp.float32),
        mesh=mesh,
        compiler_params=pltpu.CompilerParams(use_tc_tiling_on_sc=False),
        scratch_types=[
            pltpu.VMEM((4 * bpw, HQ), jnp.int32),
            pltpu.VMEM((NBUF, HV, D), jnp.float32),
            pltpu.VMEM((bpw, D), jnp.float32),
        ] + [pltpu.SemaphoreType.DMA] * NBUF,
    )
    def run(idx_hbm, tab_hbm, out_hbm, idx_v, rows_v, out_v, *sems):
        wid = lax.axis_index("s") * info.num_cores + lax.axis_index("c")
        base = wid * bpw
        pltpu.sync_copy(idx_hbm.at[pl.ds(base * 4, 4 * bpw)], idx_v)

        rows = tuple(rows_v.at[k] for k in range(NBUF))

        def fire(b, k):
            for i in range(NV):
                iv = idx_v[4 * b + i // 4, pl.ds(L * (i % 4), L)]
                pltpu.async_copy(
                    tab_hbm.at[iv],
                    rows[k].at[pl.ds(L * i, L)],
                    sems[k],
                )

        def drain(k):
            pltpu.make_async_copy(tab_hbm.at[pl.ds(0, HV)], rows[k], sems[k]).wait()

        def accum(b, rref):
            def rbody(g, acc):
                for j in range(8):
                    r = g * 8 + j
                    acc = tuple(
                        acc[d] + rref[r, pl.ds(L * d, L)] for d in range(ND)
                    )
                return acc

            acc = lax.fori_loop(
                0, H // 8, rbody,
                tuple(jnp.zeros((L,), jnp.float32) for _ in range(ND)),
            )
            for d in range(ND):
                out_v[b, pl.ds(L * d, L)] = acc[d]

        for k in range(NBUF - 1):
            fire(k, k)

        nfull = bpw // NBUF

        def body(g, carry):
            b0 = NBUF * g
            for k in range(NBUF):
                b = b0 + k

                @pl.when(b + NBUF - 1 < bpw)
                def _(b=b, k=k):
                    fire(b + NBUF - 1, (k + NBUF - 1) % NBUF)

                drain(k)
                accum(b, rows[k])
            return carry

        lax.fori_loop(0, nfull, body, 0)
        for k in range(bpw - NBUF * nfull):
            drain(k)
            accum(NBUF * nfull + k, rows[k])

        pltpu.sync_copy(out_v, out_hbm.at[pl.ds(base, bpw)])

    return run(idx4, table)
